# baseline (speedup 1.0000x reference)
import jax
import jax.numpy as jnp
from jax.experimental import pallas as pl
from jax.experimental.pallas import tpu as pltpu

B, N, D, G, K = 4, 1024, 128, 4, 3


def _body(e1_ref, e2_ref, x_ref, w1_ref, b1_ref, w2_ref, b2_ref, o_ref):
    a = (e1_ref[0, 0] + e2_ref[0, 0]).astype(jnp.bfloat16)
    agg = jnp.dot(a, x_ref[0], preferred_element_type=jnp.float32)
    h = jnp.dot(agg, w1_ref[0], preferred_element_type=jnp.float32)
    h = jnp.maximum(h + b1_ref[0], 0.0)
    agg = jnp.dot(a, h.astype(jnp.bfloat16), preferred_element_type=jnp.float32)
    h = jnp.dot(agg, w2_ref[0], preferred_element_type=jnp.float32)
    o_ref[0] = jnp.maximum(h + b2_ref[0], 0.0)


def kernel(t, x, E, W1, b1, W2, b2, interpret=False):
    et = jnp.transpose(E, (0, 3, 1, 2))
    xb = x.astype(jnp.bfloat16)
    b1r = b1.reshape(G, 1, D)
    b2r = b2.reshape(G, 1, D)
    return pl.pallas_call(
        _body,
        grid=(B,),
        in_specs=[
            pl.BlockSpec((1, 1, N, N), lambda b: (b, 1, 0, 0)),
            pl.BlockSpec((1, 1, N, N), lambda b: (b, 2, 0, 0)),
            pl.BlockSpec((1, N, D), lambda b: (b, 0, 0)),
            pl.BlockSpec((1, D, D), lambda b: (b % G, 0, 0)),
            pl.BlockSpec((1, 1, D), lambda b: (b % G, 0, 0)),
            pl.BlockSpec((1, D, D), lambda b: (b % G, 0, 0)),
            pl.BlockSpec((1, 1, D), lambda b: (b % G, 0, 0)),
        ],
        out_specs=pl.BlockSpec((1, N, D), lambda b: (b, 0, 0)),
        out_shape=jax.ShapeDtypeStruct((B, N, D), jnp.float32),
        compiler_params=pltpu.CompilerParams(
            dimension_semantics=("arbitrary",),
        ),
        interpret=interpret,
    )(et, et, xb, W1, b1r, W2, b2r)
